# baseline (speedup 1.0000x reference)
import jax
import jax.numpy as jnp
import numpy as np
from jax.experimental import pallas as pl
from jax.experimental.pallas import tpu as pltpu

B, H, S, D = 8, 16, 4096, 128
SUB = 64
N_CHUNKS = S // SUB
N_SEL = 2048 // SUB
SCALE = 1.0 / np.sqrt(D)


def _score_kernel(q_ref, k_ref, ts_ref, sel_ref):
    q = q_ref[0, 0, 0, :]
    k = k_ref[0, 0, :, :]
    t = jnp.sum(k * q[None, :], axis=1)
    ts_ref[:, 0, :] = t.reshape(N_CHUNKS, SUB) * SCALE

    qb = q.astype(jnp.bfloat16).astype(jnp.float32)
    kc = k.reshape(N_CHUNKS, SUB, D)
    kmaxb = kc.max(axis=1).astype(jnp.bfloat16).astype(jnp.float32)
    kminb = kc.min(axis=1).astype(jnp.bfloat16).astype(jnp.float32)
    s_max = jnp.sum(kmaxb * qb[None, :], axis=1)
    s_min = jnp.sum(kminb * qb[None, :], axis=1)
    cs = jnp.maximum(s_max, s_min)

    ci = cs[:, None]
    cj = cs[None, :]
    ii = jax.lax.broadcasted_iota(jnp.int32, (N_CHUNKS, N_CHUNKS), 0)
    jj = jax.lax.broadcasted_iota(jnp.int32, (N_CHUNKS, N_CHUNKS), 1)
    beats = (cj > ci) | ((cj == ci) & (jj < ii))
    bf = beats.astype(jnp.float32)
    ones_col = jnp.ones((N_CHUNKS, 1), jnp.float32)
    rank = jax.lax.dot_general(bf, ones_col, (((1,), (0,)), ((), ())),
                               preferred_element_type=jnp.float32)
    rr = jax.lax.broadcasted_iota(jnp.int32, (N_CHUNKS, N_SEL), 1).astype(jnp.float32)
    onehot = (rank == rr).astype(jnp.float32)
    idx_row = jax.lax.broadcasted_iota(jnp.int32, (1, N_CHUNKS), 1).astype(jnp.float32)
    sel = jax.lax.dot_general(idx_row, onehot, (((1,), (0,)), ((), ())),
                              preferred_element_type=jnp.float32)
    sel_ref[0, 0, 0, :] = sel[0].astype(jnp.int32)


def _attn_kernel(sel_ref, *refs):
    srefs = refs[:N_SEL]
    vrefs = refs[N_SEL:2 * N_SEL]
    out_ref = refs[2 * N_SEL]
    s = jnp.stack([srefs[j][0, 0, :] for j in range(N_SEL)], axis=0)
    m = jnp.max(s)
    p = jnp.exp(s - m)
    denom = jnp.sum(p)
    pt = jnp.transpose(p)
    acc = pt[:, 0:1] * vrefs[0][0, 0, :, :]
    for j in range(1, N_SEL):
        acc = acc + pt[:, j:j + 1] * vrefs[j][0, 0, :, :]
    out_ref[0, 0, 0, :] = jnp.sum(acc, axis=0) / denom


def _make_s_spec(j):
    return pl.BlockSpec(
        (1, 1, SUB),
        lambda b, h, sel, j=j: ((b * H + h) * N_CHUNKS + sel[b, h, j], 0, 0))


def _make_v_spec(j):
    return pl.BlockSpec((1, 1, SUB, D), lambda b, h, sel, j=j: (b, h, sel[b, h, j], 0))


@jax.jit
def kernel(q, k_cache, v_cache):
    ts, sel = pl.pallas_call(
        _score_kernel,
        grid=(B, H),
        in_specs=[
            pl.BlockSpec((1, 1, 1, D), lambda b, h: (b, h, 0, 0)),
            pl.BlockSpec((1, 1, S, D), lambda b, h: (b, h, 0, 0)),
        ],
        out_specs=[
            pl.BlockSpec((N_CHUNKS, 1, SUB), lambda b, h: (b * H + h, 0, 0)),
            pl.BlockSpec((1, 1, 1, N_SEL), lambda b, h: (b, h, 0, 0)),
        ],
        out_shape=[
            jax.ShapeDtypeStruct((B * H * N_CHUNKS, 1, SUB), jnp.float32),
            jax.ShapeDtypeStruct((B, H, 1, N_SEL), jnp.int32),
        ],
    )(q.reshape(B, H, 1, D), k_cache)

    grid_spec = pltpu.PrefetchScalarGridSpec(
        num_scalar_prefetch=1,
        grid=(B, H),
        in_specs=[_make_s_spec(j) for j in range(N_SEL)]
        + [_make_v_spec(j) for j in range(N_SEL)],
        out_specs=pl.BlockSpec((1, 1, 1, D), lambda b, h, sel: (b, h, 0, 0)),
    )
    out = pl.pallas_call(
        _attn_kernel,
        grid_spec=grid_spec,
        out_shape=jax.ShapeDtypeStruct((B, H, 1, D), jnp.float32),
    )(sel.reshape(B, H, N_SEL), *([ts] * N_SEL), *([v_cache] * N_SEL))
    return out.reshape(B, H, D)


# speedup vs baseline: 1.5066x; 1.5066x over previous
import jax
import jax.numpy as jnp
import numpy as np
from jax.experimental import pallas as pl
from jax.experimental.pallas import tpu as pltpu

B, H, S, D = 8, 16, 4096, 128
SUB = 64
N_CHUNKS = S // SUB
N_SEL = 2048 // SUB
SCALE = 1.0 / np.sqrt(D)


def _score_kernel(q_ref, k_ref, ts_ref, sel_ref, oh_ref):
    q = q_ref[0, 0, 0, :]
    k = k_ref[0, 0, :, :]
    t = jnp.sum(k * q[None, :], axis=1)
    ts_ref[0, 0, :, :] = t.reshape(N_CHUNKS, SUB) * SCALE

    qb = q.astype(jnp.bfloat16).astype(jnp.float32)
    kc = k.reshape(N_CHUNKS, SUB, D)
    kmaxb = kc.max(axis=1).astype(jnp.bfloat16).astype(jnp.float32)
    kminb = kc.min(axis=1).astype(jnp.bfloat16).astype(jnp.float32)
    s_max = jnp.sum(kmaxb * qb[None, :], axis=1)
    s_min = jnp.sum(kminb * qb[None, :], axis=1)
    cs_row = jnp.maximum(s_max, s_min)[None, :]
    cs_col = jnp.transpose(cs_row)

    ii = jax.lax.broadcasted_iota(jnp.int32, (N_CHUNKS, N_CHUNKS), 0)
    jj = jax.lax.broadcasted_iota(jnp.int32, (N_CHUNKS, N_CHUNKS), 1)
    beats = (cs_row > cs_col) | ((cs_row == cs_col) & (jj < ii))
    bf = beats.astype(jnp.float32)
    ones_col = jnp.ones((N_CHUNKS, 1), jnp.float32)
    rank = jax.lax.dot_general(bf, ones_col, (((1,), (0,)), ((), ())),
                               preferred_element_type=jnp.float32)
    rr = jax.lax.broadcasted_iota(jnp.int32, (N_CHUNKS, N_SEL), 1).astype(jnp.float32)
    onehot = (rank == rr).astype(jnp.float32)
    oh_ref[0, 0, :, :] = onehot
    idx_row = jax.lax.broadcasted_iota(jnp.int32, (1, N_CHUNKS), 1).astype(jnp.float32)
    sel = jax.lax.dot_general(idx_row, onehot, (((1,), (0,)), ((), ())),
                              preferred_element_type=jnp.float32)
    base = (pl.program_id(0) * H + pl.program_id(1)) * N_CHUNKS
    sel_ref[0, 0, 0, :] = sel[0].astype(jnp.int32) + base


def _attn_kernel(sel_ref, ts_ref, oh_ref, *vrefs_out):
    vrefs = vrefs_out[:N_SEL]
    out_ref = vrefs_out[N_SEL]
    ts2 = ts_ref[0, 0, :, :]
    oh = oh_ref[0, 0, :, :]
    s = jax.lax.dot_general(oh, ts2, (((0,), (0,)), ((), ())),
                            precision=jax.lax.Precision.HIGHEST,
                            preferred_element_type=jnp.float32)
    m = jnp.max(s)
    p = jnp.exp(s - m)
    denom = jnp.sum(p)
    pt = jnp.transpose(p)
    acc = pt[:, 0:1] * vrefs[0][0, :, :]
    for j in range(1, N_SEL):
        acc = acc + pt[:, j:j + 1] * vrefs[j][0, :, :]
    out_ref[0, 0, 0, :] = jnp.sum(acc, axis=0) / denom


def _make_v_spec(j):
    return pl.BlockSpec((1, SUB, D), lambda b, h, sel, j=j: (sel[b, h, j], 0, 0))


@jax.jit
def kernel(q, k_cache, v_cache):
    ts, sel, oh = pl.pallas_call(
        _score_kernel,
        grid=(B, H),
        in_specs=[
            pl.BlockSpec((1, 1, 1, D), lambda b, h: (b, h, 0, 0)),
            pl.BlockSpec((1, 1, S, D), lambda b, h: (b, h, 0, 0)),
        ],
        out_specs=[
            pl.BlockSpec((1, 1, N_CHUNKS, SUB), lambda b, h: (b, h, 0, 0)),
            pl.BlockSpec((1, 1, 1, N_SEL), lambda b, h: (b, h, 0, 0)),
            pl.BlockSpec((1, 1, N_CHUNKS, N_SEL), lambda b, h: (b, h, 0, 0)),
        ],
        out_shape=[
            jax.ShapeDtypeStruct((B, H, N_CHUNKS, SUB), jnp.float32),
            jax.ShapeDtypeStruct((B, H, 1, N_SEL), jnp.int32),
            jax.ShapeDtypeStruct((B, H, N_CHUNKS, N_SEL), jnp.float32),
        ],
    )(q.reshape(B, H, 1, D), k_cache)

    grid_spec = pltpu.PrefetchScalarGridSpec(
        num_scalar_prefetch=1,
        grid=(B, H),
        in_specs=[
            pl.BlockSpec((1, 1, N_CHUNKS, SUB), lambda b, h, sel: (b, h, 0, 0)),
            pl.BlockSpec((1, 1, N_CHUNKS, N_SEL), lambda b, h, sel: (b, h, 0, 0)),
        ]
        + [_make_v_spec(j) for j in range(N_SEL)],
        out_specs=pl.BlockSpec((1, 1, 1, D), lambda b, h, sel: (b, h, 0, 0)),
    )
    out = pl.pallas_call(
        _attn_kernel,
        grid_spec=grid_spec,
        out_shape=jax.ShapeDtypeStruct((B, H, 1, D), jnp.float32),
    )(sel.reshape(B, H, N_SEL), ts, oh,
      *([v_cache.reshape(B * H * N_CHUNKS, SUB, D)] * N_SEL))
    return out.reshape(B, H, D)


# speedup vs baseline: 1.7557x; 1.1654x over previous
import jax
import jax.numpy as jnp
import numpy as np
from jax.experimental import pallas as pl

B, H, S, D = 8, 16, 4096, 128
SUB = 64
N_CHUNKS = S // SUB
N_SEL = 2048 // SUB
HPB = 2
SCALE = 1.0 / np.sqrt(D)
_HI = jax.lax.Precision.HIGHEST


def _score_kernel(q_ref, k_ref, ts_ref, cs_ref, csm_ref):
    for hh in range(HPB):
        q = q_ref[0, hh, 0, :]
        kc = k_ref[0, hh]
        t2 = jnp.sum(kc * q[None, None, :], axis=2)
        ts_ref[0, hh] = t2 * SCALE

        qb = q.astype(jnp.bfloat16).astype(jnp.float32)
        kmaxb = kc.max(axis=1).astype(jnp.bfloat16).astype(jnp.float32)
        kminb = kc.min(axis=1).astype(jnp.bfloat16).astype(jnp.float32)
        s_max = jnp.sum(kmaxb * qb[None, :], axis=1)
        s_min = jnp.sum(kminb * qb[None, :], axis=1)
        cs = jnp.maximum(s_max, s_min)
        cs_ref[0, hh, 0, :] = cs
        csm_ref[0, hh] = jnp.broadcast_to(cs[None, :], (N_CHUNKS, N_CHUNKS))


def _attn_kernel(ts_ref, cs_ref, csm_ref, v_ref, out_ref):
    ii = jax.lax.broadcasted_iota(jnp.int32, (N_CHUNKS, N_CHUNKS), 0)
    jj = jax.lax.broadcasted_iota(jnp.int32, (N_CHUNKS, N_CHUNKS), 1)
    eye = (ii == jj).astype(jnp.float32)
    ones_col = jnp.ones((N_CHUNKS, 1), jnp.float32)
    for hh in range(HPB):
        ts2 = ts_ref[0, hh]
        cs = cs_ref[0, hh]
        amat = csm_ref[0, hh]
        cs_col = jax.lax.dot_general(eye, cs, (((1,), (1,)), ((), ())),
                                     precision=_HI,
                                     preferred_element_type=jnp.float32)
        m = jnp.max(ts2)
        e = jnp.exp(ts2 - m)
        beats = ((amat > cs_col) | ((amat == cs_col) & (jj < ii))) & (jj != ii)
        rank = jax.lax.dot_general(beats.astype(jnp.float32), ones_col,
                                   (((1,), (0,)), ((), ())),
                                   preferred_element_type=jnp.float32)
        mask_col = (rank < float(N_SEL)).astype(jnp.float32)
        p = e * mask_col
        denom = jnp.sum(p)
        pt = jax.lax.dot_general(p, eye, (((0,), (0,)), ((), ())),
                                 precision=_HI,
                                 preferred_element_type=jnp.float32)
        vc = v_ref[0, hh]
        acc = pt[:, 0:1] * vc[0]
        for c in range(1, N_CHUNKS):
            acc = acc + pt[:, c:c + 1] * vc[c]
        out_ref[0, hh, 0, :] = jnp.sum(acc, axis=0) / denom


@jax.jit
def kernel(q, k_cache, v_cache):
    k5 = k_cache.reshape(B, H, N_CHUNKS, SUB, D)
    v5 = v_cache.reshape(B, H, N_CHUNKS, SUB, D)
    ts, cs, csm = pl.pallas_call(
        _score_kernel,
        grid=(B, H // HPB),
        in_specs=[
            pl.BlockSpec((1, HPB, 1, D), lambda b, h: (b, h, 0, 0)),
            pl.BlockSpec((1, HPB, N_CHUNKS, SUB, D), lambda b, h: (b, h, 0, 0, 0)),
        ],
        out_specs=[
            pl.BlockSpec((1, HPB, N_CHUNKS, SUB), lambda b, h: (b, h, 0, 0)),
            pl.BlockSpec((1, HPB, 1, N_CHUNKS), lambda b, h: (b, h, 0, 0)),
            pl.BlockSpec((1, HPB, N_CHUNKS, N_CHUNKS), lambda b, h: (b, h, 0, 0)),
        ],
        out_shape=[
            jax.ShapeDtypeStruct((B, H, N_CHUNKS, SUB), jnp.float32),
            jax.ShapeDtypeStruct((B, H, 1, N_CHUNKS), jnp.float32),
            jax.ShapeDtypeStruct((B, H, N_CHUNKS, N_CHUNKS), jnp.float32),
        ],
    )(q.reshape(B, H, 1, D), k5)

    out = pl.pallas_call(
        _attn_kernel,
        grid=(B, H // HPB),
        in_specs=[
            pl.BlockSpec((1, HPB, N_CHUNKS, SUB), lambda b, h: (b, h, 0, 0)),
            pl.BlockSpec((1, HPB, 1, N_CHUNKS), lambda b, h: (b, h, 0, 0)),
            pl.BlockSpec((1, HPB, N_CHUNKS, N_CHUNKS), lambda b, h: (b, h, 0, 0)),
            pl.BlockSpec((1, HPB, N_CHUNKS, SUB, D), lambda b, h: (b, h, 0, 0, 0)),
        ],
        out_specs=pl.BlockSpec((1, HPB, 1, D), lambda b, h: (b, h, 0, 0)),
        out_shape=jax.ShapeDtypeStruct((B, H, 1, D), jnp.float32),
    )(ts, cs, csm, v5)
    return out.reshape(B, H, D)


# speedup vs baseline: 1.7762x; 1.0117x over previous
import jax
import jax.numpy as jnp
import numpy as np
from jax.experimental import pallas as pl

B, H, S, D = 8, 16, 4096, 128
SUB = 64
N_CHUNKS = S // SUB
N_SEL = 2048 // SUB
HPB = 4
SCALE = 1.0 / np.sqrt(D)
_HI = jax.lax.Precision.HIGHEST


def _score_kernel(q_ref, k_ref, ts_ref, cs_ref, csm_ref):
    for hh in range(HPB):
        q = q_ref[0, hh, 0, :]
        kc = k_ref[0, hh]
        t2 = jnp.sum(kc * q[None, None, :], axis=2)
        ts_ref[0, hh] = t2 * SCALE

        qb = q.astype(jnp.bfloat16).astype(jnp.float32)
        kmaxb = kc.max(axis=1).astype(jnp.bfloat16).astype(jnp.float32)
        kminb = kc.min(axis=1).astype(jnp.bfloat16).astype(jnp.float32)
        s_max = jnp.sum(kmaxb * qb[None, :], axis=1)
        s_min = jnp.sum(kminb * qb[None, :], axis=1)
        cs = jnp.maximum(s_max, s_min)
        cs_ref[0, hh, 0, :] = cs
        csm_ref[0, hh] = jnp.broadcast_to(cs[None, :], (N_CHUNKS, N_CHUNKS))


def _attn_kernel(ts_ref, cs_ref, csm_ref, v_ref, out_ref):
    ii = jax.lax.broadcasted_iota(jnp.int32, (N_CHUNKS, N_CHUNKS), 0)
    jj = jax.lax.broadcasted_iota(jnp.int32, (N_CHUNKS, N_CHUNKS), 1)
    eye = (ii == jj).astype(jnp.float32)
    ones_col = jnp.ones((N_CHUNKS, 1), jnp.float32)
    for hh in range(HPB):
        ts2 = ts_ref[0, hh]
        cs = cs_ref[0, hh]
        amat = csm_ref[0, hh]
        cs_col = jax.lax.dot_general(eye, cs, (((1,), (1,)), ((), ())),
                                     precision=_HI,
                                     preferred_element_type=jnp.float32)
        m = jnp.max(ts2)
        e = jnp.exp(ts2 - m)
        beats = ((amat > cs_col) | ((amat == cs_col) & (jj < ii))) & (jj != ii)
        rank = jax.lax.dot_general(beats.astype(jnp.float32), ones_col,
                                   (((1,), (0,)), ((), ())),
                                   preferred_element_type=jnp.float32)
        mask_col = (rank < float(N_SEL)).astype(jnp.float32)
        p = e * mask_col
        denom = jnp.sum(p)
        pt = jax.lax.dot_general(p, eye, (((0,), (0,)), ((), ())),
                                 precision=_HI,
                                 preferred_element_type=jnp.float32)
        vc = v_ref[0, hh]
        acc = pt[:, 0:1] * vc[0]
        for c in range(1, N_CHUNKS):
            acc = acc + pt[:, c:c + 1] * vc[c]
        out_ref[0, hh, 0, :] = jnp.sum(acc, axis=0) / denom


@jax.jit
def kernel(q, k_cache, v_cache):
    k5 = k_cache.reshape(B, H, N_CHUNKS, SUB, D)
    v5 = v_cache.reshape(B, H, N_CHUNKS, SUB, D)
    ts, cs, csm = pl.pallas_call(
        _score_kernel,
        grid=(B, H // HPB),
        in_specs=[
            pl.BlockSpec((1, HPB, 1, D), lambda b, h: (b, h, 0, 0)),
            pl.BlockSpec((1, HPB, N_CHUNKS, SUB, D), lambda b, h: (b, h, 0, 0, 0)),
        ],
        out_specs=[
            pl.BlockSpec((1, HPB, N_CHUNKS, SUB), lambda b, h: (b, h, 0, 0)),
            pl.BlockSpec((1, HPB, 1, N_CHUNKS), lambda b, h: (b, h, 0, 0)),
            pl.BlockSpec((1, HPB, N_CHUNKS, N_CHUNKS), lambda b, h: (b, h, 0, 0)),
        ],
        out_shape=[
            jax.ShapeDtypeStruct((B, H, N_CHUNKS, SUB), jnp.float32),
            jax.ShapeDtypeStruct((B, H, 1, N_CHUNKS), jnp.float32),
            jax.ShapeDtypeStruct((B, H, N_CHUNKS, N_CHUNKS), jnp.float32),
        ],
    )(q.reshape(B, H, 1, D), k5)

    out = pl.pallas_call(
        _attn_kernel,
        grid=(B, H // HPB),
        in_specs=[
            pl.BlockSpec((1, HPB, N_CHUNKS, SUB), lambda b, h: (b, h, 0, 0)),
            pl.BlockSpec((1, HPB, 1, N_CHUNKS), lambda b, h: (b, h, 0, 0)),
            pl.BlockSpec((1, HPB, N_CHUNKS, N_CHUNKS), lambda b, h: (b, h, 0, 0)),
            pl.BlockSpec((1, HPB, N_CHUNKS, SUB, D), lambda b, h: (b, h, 0, 0, 0)),
        ],
        out_specs=pl.BlockSpec((1, HPB, 1, D), lambda b, h: (b, h, 0, 0)),
        out_shape=jax.ShapeDtypeStruct((B, H, 1, D), jnp.float32),
    )(ts, cs, csm, v5)
    return out.reshape(B, H, D)
